# baseline (speedup 1.0000x reference)
import functools
import jax
import jax.numpy as jnp
from jax import lax
from jax.experimental import pallas as pl
from jax.experimental.pallas import tpu as pltpu
from jax.experimental.pallas import tpu_sc as plsc

_B = 16384
_D = 64
_NC = 2
_NS = 16
_NW = _NC * _NS
_BPW = _B // _NW
_CHUNK = 128
_NCHUNK = _BPW // _CHUNK
_L = 16
_NG = _BPW // _L


def _sc_body(feat2_hbm, pairidx_hbm, colbase_hbm, cent2_hbm, out_hbm,
             idx_v, col_v, rows_v, feat_v, acc_v, gsem):
    wid = lax.axis_index("s") * _NC + lax.axis_index("c")
    base = wid * _BPW

    pltpu.sync_copy(pairidx_hbm.at[wid], idx_v)
    pltpu.sync_copy(colbase_hbm.at[pl.ds(base, _BPW)], col_v)

    copies = []
    for j in range(_NCHUNK):
        copies.append(
            pltpu.async_copy(
                cent2_hbm.at[idx_v.at[j]],
                rows_v.at[pl.ds(j * _CHUNK, _CHUNK)],
                gsem,
            )
        )
    pltpu.sync_copy(feat2_hbm.at[pl.ds(wid * (_BPW // 2), _BPW // 2)], feat_v)
    for c in copies:
        c.wait()

    lane = lax.iota(jnp.int32, _L)
    fcol0 = (lane % 2) * _D
    frow0 = lane // 2

    def group_body(g, accs):
        rows0 = g * _L + lane
        frows = g * (_L // 2) + frow0
        colbase = col_v[pl.ds(g * _L, _L)]
        accs = list(accs)
        for d in range(_D):
            dsplat = jnp.full((_L,), d, jnp.int32)
            f = plsc.load_gather(feat_v, [frows, fcol0 + dsplat])
            ce = plsc.load_gather(rows_v, [rows0, colbase + d])
            df = f - ce
            accs[d % 4] = accs[d % 4] + df * df
        return tuple(accs)

    z = jnp.zeros((_L,), jnp.float32)
    a0, a1, a2, a3 = lax.fori_loop(0, _NG, group_body, (z, z, z, z))
    acc_v[...] = (a0 + a1 + a2 + a3) * jnp.float32(1.0 / (_B * _D))
    pltpu.sync_copy(acc_v, out_hbm.at[wid])


@jax.jit
def _center_loss_sc(features2, pair_idx, colbase, centers2):
    mesh = plsc.VectorSubcoreMesh(
        core_axis_name="c", subcore_axis_name="s",
        num_cores=_NC, num_subcores=_NS,
    )
    partials = pl.kernel(
        _sc_body,
        out_type=jax.ShapeDtypeStruct((_NW, _L), jnp.float32),
        mesh=mesh,
        scratch_types=[
            pltpu.VMEM((_NCHUNK, _CHUNK), jnp.int32),
            pltpu.VMEM((_BPW,), jnp.int32),
            pltpu.VMEM((_BPW, 2 * _D), jnp.float32),
            pltpu.VMEM((_BPW // 2, 2 * _D), jnp.float32),
            pltpu.VMEM((_L,), jnp.float32),
            pltpu.SemaphoreType.DMA,
        ],
        compiler_params=pltpu.CompilerParams(needs_layout_passes=False),
    )(features2, pair_idx, colbase, centers2)
    return jnp.sum(partials)


def kernel(features, labels, centers):
    labels = labels.astype(jnp.int32)
    pair_idx = (labels // 2).reshape(_NW, _NCHUNK, _CHUNK)
    colbase = (labels % 2) * _D
    centers2 = centers.reshape(centers.shape[0] // 2, 2 * _D)
    features2 = features.reshape(_B // 2, 2 * _D)
    return _center_loss_sc(features2, pair_idx, colbase, centers2)


# speedup vs baseline: 1.8427x; 1.8427x over previous
import functools
import jax
import jax.numpy as jnp
from jax import lax
from jax.experimental import pallas as pl
from jax.experimental.pallas import tpu as pltpu
from jax.experimental.pallas import tpu_sc as plsc

_B = 16384
_D = 64
_NC = 2
_NS = 16
_NW = _NC * _NS
_BPW = _B // _NW
_C = 128
_NCHUNK = _BPW // _C
_L = 16
_NG = _C // _L


def _sc_body(feat_hbm, lab_hbm, cent_hbm, out_hbm,
             lab_v, rows_v, feat_v, acc_v, gsem, fsem):
    wid = lax.axis_index("s") * _NC + lax.axis_index("c")
    base = wid * _BPW

    pltpu.sync_copy(lab_hbm.at[wid], lab_v)

    lane = lax.iota(jnp.int32, _L)

    def fire(k):
        buf = k % 2
        fdesc = pltpu.async_copy(
            feat_hbm.at[pl.ds(base + k * _C, _C)], feat_v.at[buf], fsem)

        def issue_group(g, _):
            labs = lab_v[pl.ds(k * _C + g * _L, _L)]
            for j in range(_L):
                r = jnp.sum(jnp.where(lane == j, labs, 0))
                pltpu.async_copy(
                    cent_hbm.at[r], rows_v.at[buf, g * _L + j], gsem)
            return 0

        lax.fori_loop(0, _NG, issue_group, 0)
        return fdesc

    def drain(k, fdesc):
        buf = k % 2
        pltpu.make_async_copy(
            cent_hbm.at[pl.ds(0, _C)], rows_v.at[buf], gsem).wait()
        fdesc.wait()

    def accumulate(k, accs):
        buf = k % 2

        def row_body(r, accs):
            a = list(accs)
            for c in range(_D // _L):
                f = feat_v[buf, r, pl.ds(c * _L, _L)]
                ce = rows_v[buf, r, pl.ds(c * _L, _L)]
                df = f - ce
                a[c] = a[c] + df * df
            return tuple(a)

        return lax.fori_loop(0, _C, row_body, accs)

    zero = jnp.zeros((_L,), jnp.float32)
    accs = (zero, zero, zero, zero)

    fdescs = [fire(0), fire(1)]
    for k in range(_NCHUNK):
        drain(k, fdescs[k])
        accs = accumulate(k, accs)
        if k + 2 < _NCHUNK:
            fdescs.append(fire(k + 2))

    acc_v[...] = (accs[0] + accs[1] + accs[2] + accs[3]) * jnp.float32(
        1.0 / (_B * _D))
    pltpu.sync_copy(acc_v, out_hbm.at[wid])


@jax.jit
def _center_loss_sc(features, labels_r, centers):
    mesh = plsc.VectorSubcoreMesh(
        core_axis_name="c", subcore_axis_name="s",
        num_cores=_NC, num_subcores=_NS,
    )
    partials = pl.kernel(
        _sc_body,
        out_type=jax.ShapeDtypeStruct((_NW, _L), jnp.float32),
        mesh=mesh,
        scratch_types=[
            pltpu.VMEM((_BPW,), jnp.int32),
            pltpu.VMEM((2, _C, _D), jnp.float32),
            pltpu.VMEM((2, _C, _D), jnp.float32),
            pltpu.VMEM((_L,), jnp.float32),
            pltpu.SemaphoreType.DMA,
            pltpu.SemaphoreType.DMA,
        ],
        compiler_params=pltpu.CompilerParams(needs_layout_passes=False),
    )(features, labels_r, centers)
    return jnp.sum(partials)


def kernel(features, labels, centers):
    labels_r = labels.astype(jnp.int32).reshape(_NW, _BPW)
    return _center_loss_sc(features, labels_r, centers)


# speedup vs baseline: 1.8456x; 1.0016x over previous
import functools
import jax
import jax.numpy as jnp
from jax import lax
from jax.experimental import pallas as pl
from jax.experimental.pallas import tpu as pltpu
from jax.experimental.pallas import tpu_sc as plsc

_B = 16384
_D = 64
_NC = 2
_NS = 16
_NW = _NC * _NS
_BPW = _B // _NW
_C = 128
_NCHUNK = _BPW // _C
_L = 16
_NG = _C // _L


def _sc_body(feat_hbm, lab_hbm, cent_hbm, out_hbm,
             lab_v, rows_v, feat_v, acc_v, gsem, fsem):
    wid = lax.axis_index("s") * _NC + lax.axis_index("c")
    base = wid * _BPW

    pltpu.sync_copy(lab_hbm.at[wid], lab_v)

    lane = lax.iota(jnp.int32, _L)

    def fire(k):
        buf = k % 2
        fdesc = pltpu.async_copy(
            feat_hbm.at[pl.ds(base + k * _C, _C)], feat_v.at[buf], fsem)

        def issue_group(g, _):
            labs = lab_v[pl.ds(k * _C + g * _L, _L)]
            for j in range(_L):
                r = jnp.sum(jnp.where(lane == j, labs, 0))
                pltpu.async_copy(
                    cent_hbm.at[r], rows_v.at[buf, g * _L + j], gsem)
            return 0

        lax.fori_loop(0, _NG, issue_group, 0)
        return fdesc

    def drain(k, fdesc):
        buf = k % 2
        pltpu.make_async_copy(
            cent_hbm.at[pl.ds(0, _C)], rows_v.at[buf], gsem).wait()
        fdesc.wait()

    def accumulate(k, accs):
        buf = k % 2

        def row_body(r, accs):
            a = list(accs)
            for c in range(_D // _L):
                f = feat_v[buf, r, pl.ds(c * _L, _L)]
                ce = rows_v[buf, r, pl.ds(c * _L, _L)]
                df = f - ce
                a[c] = a[c] + df * df
            return tuple(a)

        return lax.fori_loop(0, _C, row_body, accs)

    zero = jnp.zeros((_L,), jnp.float32)
    accs = (zero, zero, zero, zero)

    fdescs = [fire(0), fire(1)]
    for k in range(_NCHUNK):
        drain(k, fdescs[k])
        accs = accumulate(k, accs)
        if k + 2 < _NCHUNK:
            fdescs.append(fire(k + 2))

    acc_v[...] = (accs[0] + accs[1] + accs[2] + accs[3]) * jnp.float32(
        1.0 / (_B * _D))
    pltpu.sync_copy(acc_v, out_hbm.at[wid])


@jax.jit
def _center_loss_sc(features, labels_r, centers):
    mesh = plsc.VectorSubcoreMesh(
        core_axis_name="c", subcore_axis_name="s",
        num_cores=_NC, num_subcores=_NS,
    )
    partials = pl.kernel(
        _sc_body,
        out_type=jax.ShapeDtypeStruct((_NW, _L), jnp.float32),
        mesh=mesh,
        scratch_types=[
            pltpu.VMEM((_BPW,), jnp.int32),
            pltpu.VMEM((2, _C, _D), jnp.float32),
            pltpu.VMEM((2, _C, _D), jnp.float32),
            pltpu.VMEM((_L,), jnp.float32),
            pltpu.SemaphoreType.DMA,
            pltpu.SemaphoreType.DMA,
        ],
        compiler_params=pltpu.CompilerParams(
            needs_layout_passes=False,
            disable_bounds_checks=True,
            disable_semaphore_checks=True,
        ),
    )(features, labels_r, centers)
    return jnp.sum(partials)


def kernel(features, labels, centers):
    labels_r = labels.astype(jnp.int32).reshape(_NW, _BPW)
    return _center_loss_sc(features, labels_r, centers)
